# baseline (speedup 1.0000x reference)
import functools

import jax
import jax.numpy as jnp
from jax import lax
from jax.experimental import pallas as pl
from jax.experimental.pallas import tpu as pltpu
from jax.experimental.pallas import tpu_sc as plsc

N = 4096
F = 512
KP = 1024


def _head_body(x_ref, w_ref, s_ref, feat_ref, idx_ref):
    w = w_ref[...]
    nrm = jax.lax.rsqrt(jnp.maximum(jnp.sum(w * w), 1e-12))
    y = jnp.dot(x_ref[...], w * nrm,
                preferred_element_type=jnp.float32)
    feat_ref[:, :F] = x_ref[...] * jnp.tanh(y)
    sbc = lax.bitcast_convert_type(s_ref[...], jnp.float32)
    feat_ref[:, F:] = jnp.broadcast_to(sbc, (N, 128))

    srow = y.T
    cw = 512
    rank = jnp.zeros((1, N), jnp.float32)
    ii = lax.broadcasted_iota(jnp.int32, (cw, N), 1)
    for cj in range(N // cw):
        sj = y[cj * cw:(cj + 1) * cw, :]
        jj = lax.broadcasted_iota(jnp.int32, (cw, N), 0) + cj * cw
        before = (sj > srow) | ((sj == srow) & (jj < ii))
        rank = rank + jnp.sum(before.astype(jnp.float32), axis=0,
                              keepdims=True)
    maskr = (rank < KP).astype(jnp.float32)
    lane = lax.broadcasted_iota(jnp.int32, (1, N), 1)
    crow = maskr
    sh = 1
    while sh < N:
        r = pltpu.roll(crow, sh, axis=1)
        crow = crow + jnp.where(lane >= sh, r, 0.0)
        sh *= 2
    for pi in range(KP // cw):
        pp = (lax.broadcasted_iota(jnp.int32, (cw, N), 0) + pi * cw
              ).astype(jnp.float32)
        cnt = jnp.sum((crow <= pp).astype(jnp.float32), axis=1, keepdims=True)
        idx_ref[pl.ds(pi * cw, cw), :] = cnt.astype(jnp.int32)


def _head(X, w, S):
    return pl.pallas_call(
        _head_body,
        out_shape=(
            jax.ShapeDtypeStruct((N, F + 128), jnp.float32),
            jax.ShapeDtypeStruct((KP, 1), jnp.int32),
        ),
    )(X, w, jnp.reshape(S, (N, 1)))


_TRG = 4
_TBJ = 2048


def _tr_body(a1_ref, a2_ref, o_ref):
    lo = a1_ref[...].T.astype(jnp.bfloat16)
    hi = a2_ref[...].T.astype(jnp.bfloat16)
    lo32 = lax.convert_element_type(
        lax.bitcast_convert_type(lo, jnp.uint16), jnp.uint32)
    hi32 = lax.convert_element_type(
        lax.bitcast_convert_type(hi, jnp.uint16), jnp.uint32)
    o_ref[...] = ((hi32 << 16) | lo32).astype(jnp.int32)


def _transpose_packed(A):
    kb = N // 2 // _TRG
    return pl.pallas_call(
        _tr_body,
        grid=(_TRG, N // _TBJ),
        in_specs=[
            pl.BlockSpec((kb, _TBJ), lambda g, j: (g, j)),
            pl.BlockSpec((kb, _TBJ), lambda g, j: (g + _TRG, j)),
        ],
        out_specs=pl.BlockSpec((_TBJ, kb), lambda g, j: (j, g)),
        out_shape=jax.ShapeDtypeStruct((N, N // 2), jnp.int32),
    )(A, A)


_NC = 2
_NS = 16
_NW = _NC * _NS
_BPW = KP // _NW
_AC = 8


def _sc_gather_a_body(feat_hbm, a_hbm, idx_hbm, idx2_hbm,
                      xp_out, sp_out, ar_out,
                      idx_v, idxc_v, xbuf, ab0, ab1, semx, sem0, sem1):
    wid = lax.axis_index("s") * _NC + lax.axis_index("c")
    base = wid * _BPW
    pltpu.sync_copy(idx_hbm.at[pl.ds(base, _BPW)], idx_v)
    pltpu.sync_copy(idx2_hbm.at[pl.ds(wid * (_BPW // _AC), _BPW // _AC)],
                    idxc_v)
    cpx = pltpu.async_copy(feat_hbm.at[idx_v], xbuf, semx)
    bufs = (ab0, ab1)
    sems = (sem0, sem1)
    nc = _BPW // _AC
    cps = [pltpu.async_copy(a_hbm.at[idxc_v.at[0]], ab0, sem0)]
    for c in range(nc):
        if c + 1 < nc:
            cps.append(pltpu.async_copy(a_hbm.at[idxc_v.at[c + 1]],
                                        bufs[(c + 1) % 2], sems[(c + 1) % 2]))
        cps[c].wait()
        pltpu.sync_copy(bufs[c % 2], ar_out.at[pl.ds(base + c * _AC, _AC)])
    cpx.wait()
    pltpu.sync_copy(xbuf.at[:, :F], xp_out.at[pl.ds(base, _BPW)])
    pltpu.sync_copy(xbuf.at[:, F:], sp_out.at[pl.ds(base, _BPW)])


def _sc_gather_a(feat, A, idx, idx2):
    mesh = plsc.VectorSubcoreMesh(core_axis_name="c", subcore_axis_name="s")
    run = functools.partial(
        pl.kernel,
        mesh=mesh,
        out_type=[
            jax.ShapeDtypeStruct((KP, F), jnp.float32),
            jax.ShapeDtypeStruct((KP, 128), jnp.float32),
            jax.ShapeDtypeStruct((KP, N), jnp.float32),
        ],
        scratch_types=[
            pltpu.VMEM((_BPW,), jnp.int32),
            pltpu.VMEM((_BPW // _AC, _AC), jnp.int32),
            pltpu.VMEM((_BPW, F + 128), jnp.float32),
            pltpu.VMEM((_AC, N), jnp.float32),
            pltpu.VMEM((_AC, N), jnp.float32),
            pltpu.SemaphoreType.DMA,
            pltpu.SemaphoreType.DMA,
            pltpu.SemaphoreType.DMA,
        ],
    )(_sc_gather_a_body)
    return run(feat, A, idx, idx2)


_ACT = 16


def _sc_gather_at_body(at_hbm, idx2_hbm, atr_out, idxc_v, ab0, ab1,
                       sem0, sem1):
    wid = lax.axis_index("s") * _NC + lax.axis_index("c")
    base = wid * _BPW
    pltpu.sync_copy(idx2_hbm.at[pl.ds(wid * (_BPW // _ACT), _BPW // _ACT)],
                    idxc_v)
    bufs = (ab0, ab1)
    sems = (sem0, sem1)
    nc = _BPW // _ACT
    cps = [pltpu.async_copy(at_hbm.at[idxc_v.at[0]], ab0, sem0)]
    for c in range(nc):
        if c + 1 < nc:
            cps.append(pltpu.async_copy(at_hbm.at[idxc_v.at[c + 1]],
                                        bufs[(c + 1) % 2], sems[(c + 1) % 2]))
        cps[c].wait()
        pltpu.sync_copy(bufs[c % 2], atr_out.at[pl.ds(base + c * _ACT, _ACT)])


def _sc_gather_at(At, idx2t):
    mesh = plsc.VectorSubcoreMesh(core_axis_name="c", subcore_axis_name="s")
    run = functools.partial(
        pl.kernel,
        mesh=mesh,
        out_type=[jax.ShapeDtypeStruct((KP, N // 2), jnp.int32)],
        scratch_types=[
            pltpu.VMEM((_BPW // _ACT, _ACT), jnp.int32),
            pltpu.VMEM((_ACT, N // 2), jnp.int32),
            pltpu.VMEM((_ACT, N // 2), jnp.int32),
            pltpu.SemaphoreType.DMA,
            pltpu.SemaphoreType.DMA,
        ],
    )(_sc_gather_at_body)
    return run(At, idx2t)[0]


_MB = 512


def _mm_body(ar_ref, atrp_ref, o_ref):
    w = lax.bitcast_convert_type(atrp_ref[...], jnp.uint32)
    lo = lax.bitcast_convert_type(
        lax.convert_element_type(w & 0xFFFF, jnp.uint16), jnp.bfloat16)
    hi = lax.bitcast_convert_type(
        lax.convert_element_type(w >> 16, jnp.uint16), jnp.bfloat16)
    ar = ar_ref[...]
    dn = (((1,), (1,)), ((), ()))
    o_ref[...] = (
        lax.dot_general(ar[:, :N // 2].astype(jnp.bfloat16), lo, dn,
                        preferred_element_type=jnp.float32)
        + lax.dot_general(ar[:, N // 2:].astype(jnp.bfloat16), hi, dn,
                          preferred_element_type=jnp.float32))


def _pool_matmul(Ar, Atrp):
    g = KP // _MB
    return pl.pallas_call(
        _mm_body,
        grid=(g,),
        in_specs=[
            pl.BlockSpec((_MB, N), lambda i: (i, 0)),
            pl.BlockSpec((KP, N // 2), lambda i: (0, 0)),
        ],
        out_specs=pl.BlockSpec((_MB, KP), lambda i: (i, 0)),
        out_shape=jax.ShapeDtypeStruct((KP, KP), jnp.float32),
    )(Ar, Atrp)


def kernel(X, A, S, kernel):
    feat, idx2d = _head(X, kernel, S)
    idx = jnp.reshape(idx2d, (KP,))
    idx2 = jnp.reshape(idx, (KP // _AC, _AC))
    Xp, Sp2, Ar = _sc_gather_a(feat, A, idx, idx2)
    At = _transpose_packed(A)
    Atr = _sc_gather_at(At, jnp.reshape(idx, (KP // _ACT, _ACT)))
    Ap = _pool_matmul(Ar, Atr)
    Sp = lax.bitcast_convert_type(Sp2[:, 0], jnp.int32)
    return Xp, Ap, Sp


# speedup vs baseline: 1.0033x; 1.0033x over previous
import functools

import jax
import jax.numpy as jnp
from jax import lax
from jax.experimental import pallas as pl
from jax.experimental.pallas import tpu as pltpu
from jax.experimental.pallas import tpu_sc as plsc

N = 4096
F = 512
KP = 1024


def _head_body(x_ref, w_ref, s_ref, feat_ref, idx_ref):
    w = w_ref[...]
    nrm = jax.lax.rsqrt(jnp.maximum(jnp.sum(w * w), 1e-12))
    y = jnp.dot(x_ref[...], w * nrm,
                preferred_element_type=jnp.float32)
    feat_ref[:, :F] = x_ref[...] * jnp.tanh(y)
    sbc = lax.bitcast_convert_type(s_ref[...], jnp.float32)
    feat_ref[:, F:] = jnp.broadcast_to(sbc, (N, 128))

    srow = y.T
    cw = 512
    rank = jnp.zeros((1, N), jnp.float32)
    ii = lax.broadcasted_iota(jnp.int32, (cw, N), 1)
    for cj in range(N // cw):
        sj = y[cj * cw:(cj + 1) * cw, :]
        jj = lax.broadcasted_iota(jnp.int32, (cw, N), 0) + cj * cw
        before = (sj > srow) | ((sj == srow) & (jj < ii))
        rank = rank + jnp.sum(before.astype(jnp.float32), axis=0,
                              keepdims=True)
    maskr = (rank < KP).astype(jnp.float32)
    lane = lax.broadcasted_iota(jnp.int32, (1, N), 1)
    crow = maskr
    sh = 1
    while sh < N:
        r = pltpu.roll(crow, sh, axis=1)
        crow = crow + jnp.where(lane >= sh, r, 0.0)
        sh *= 2
    for pi in range(KP // cw):
        pp = (lax.broadcasted_iota(jnp.int32, (cw, N), 0) + pi * cw
              ).astype(jnp.float32)
        cnt = jnp.sum((crow <= pp).astype(jnp.float32), axis=1, keepdims=True)
        idx_ref[pl.ds(pi * cw, cw), :] = cnt.astype(jnp.int32)


def _head(X, w, S):
    return pl.pallas_call(
        _head_body,
        out_shape=(
            jax.ShapeDtypeStruct((N, F + 128), jnp.float32),
            jax.ShapeDtypeStruct((KP, 1), jnp.int32),
        ),
    )(X, w, jnp.reshape(S, (N, 1)))


_TRG = 4
_TBJ = 2048


def _tr_body(a1_ref, a2_ref, o_ref):
    lo = a1_ref[...].T.astype(jnp.bfloat16)
    hi = a2_ref[...].T.astype(jnp.bfloat16)
    lo32 = lax.convert_element_type(
        lax.bitcast_convert_type(lo, jnp.uint16), jnp.uint32)
    hi32 = lax.convert_element_type(
        lax.bitcast_convert_type(hi, jnp.uint16), jnp.uint32)
    o_ref[...] = ((hi32 << 16) | lo32).astype(jnp.int32)


def _transpose_packed(A):
    kb = N // 2 // _TRG
    return pl.pallas_call(
        _tr_body,
        grid=(_TRG, N // _TBJ),
        in_specs=[
            pl.BlockSpec((kb, _TBJ), lambda g, j: (g, j)),
            pl.BlockSpec((kb, _TBJ), lambda g, j: (g + _TRG, j)),
        ],
        out_specs=pl.BlockSpec((_TBJ, kb), lambda g, j: (j, g)),
        out_shape=jax.ShapeDtypeStruct((N, N // 2), jnp.int32),
    )(A, A)


_NC = 2
_NS = 16
_NW = _NC * _NS
_BPW = KP // _NW
_AC = 8


def _sc_gather_a_body(feat_hbm, a_hbm, idx_hbm, idx2_hbm,
                      xp_out, sp_out, ar_out,
                      idx_v, idxc_v, xbuf, ab0, ab1, semx, sem0, sem1):
    wid = lax.axis_index("s") * _NC + lax.axis_index("c")
    base = wid * _BPW
    pltpu.sync_copy(idx_hbm.at[pl.ds(base, _BPW)], idx_v)
    pltpu.sync_copy(idx2_hbm.at[pl.ds(wid * (_BPW // _AC), _BPW // _AC)],
                    idxc_v)
    cpx = pltpu.async_copy(feat_hbm.at[idx_v], xbuf, semx)
    bufs = (ab0, ab1)
    sems = (sem0, sem1)
    nc = _BPW // _AC
    cps = [pltpu.async_copy(a_hbm.at[idxc_v.at[0]], ab0, sem0)]
    for c in range(nc):
        if c + 1 < nc:
            cps.append(pltpu.async_copy(a_hbm.at[idxc_v.at[c + 1]],
                                        bufs[(c + 1) % 2], sems[(c + 1) % 2]))
        cps[c].wait()
        pltpu.sync_copy(bufs[c % 2], ar_out.at[pl.ds(base + c * _AC, _AC)])
    cpx.wait()
    pltpu.sync_copy(xbuf.at[:, :F], xp_out.at[pl.ds(base, _BPW)])
    pltpu.sync_copy(xbuf.at[:, F:], sp_out.at[pl.ds(base, _BPW)])


def _sc_gather_a(feat, A, idx, idx2):
    mesh = plsc.VectorSubcoreMesh(core_axis_name="c", subcore_axis_name="s")
    run = functools.partial(
        pl.kernel,
        mesh=mesh,
        out_type=[
            jax.ShapeDtypeStruct((KP, F), jnp.float32),
            jax.ShapeDtypeStruct((KP, 128), jnp.float32),
            jax.ShapeDtypeStruct((KP, N), jnp.float32),
        ],
        scratch_types=[
            pltpu.VMEM((_BPW,), jnp.int32),
            pltpu.VMEM((_BPW // _AC, _AC), jnp.int32),
            pltpu.VMEM((_BPW, F + 128), jnp.float32),
            pltpu.VMEM((_AC, N), jnp.float32),
            pltpu.VMEM((_AC, N), jnp.float32),
            pltpu.SemaphoreType.DMA,
            pltpu.SemaphoreType.DMA,
            pltpu.SemaphoreType.DMA,
        ],
    )(_sc_gather_a_body)
    return run(feat, A, idx, idx2)


_ACT = 16


def _sc_gather_at_body(at_hbm, idx2_hbm, atr_out, idxc_v, ab0, ab1,
                       sem0, sem1):
    wid = lax.axis_index("s") * _NC + lax.axis_index("c")
    base = wid * _BPW
    pltpu.sync_copy(idx2_hbm.at[pl.ds(wid * (_BPW // _ACT), _BPW // _ACT)],
                    idxc_v)
    bufs = (ab0, ab1)
    sems = (sem0, sem1)
    nc = _BPW // _ACT
    cps = [pltpu.async_copy(at_hbm.at[idxc_v.at[0]], ab0, sem0)]
    for c in range(nc):
        if c + 1 < nc:
            cps.append(pltpu.async_copy(at_hbm.at[idxc_v.at[c + 1]],
                                        bufs[(c + 1) % 2], sems[(c + 1) % 2]))
        cps[c].wait()
        pltpu.sync_copy(bufs[c % 2], atr_out.at[pl.ds(base + c * _ACT, _ACT)])


def _sc_gather_at(At, idx2t):
    mesh = plsc.VectorSubcoreMesh(core_axis_name="c", subcore_axis_name="s")
    run = functools.partial(
        pl.kernel,
        mesh=mesh,
        out_type=[jax.ShapeDtypeStruct((KP, N // 2), jnp.int32)],
        scratch_types=[
            pltpu.VMEM((_BPW // _ACT, _ACT), jnp.int32),
            pltpu.VMEM((_ACT, N // 2), jnp.int32),
            pltpu.VMEM((_ACT, N // 2), jnp.int32),
            pltpu.SemaphoreType.DMA,
            pltpu.SemaphoreType.DMA,
        ],
    )(_sc_gather_at_body)
    return run(At, idx2t)[0]


_MB = 256


def _mm_body(ar_ref, atrp_ref, o_ref):
    w = lax.bitcast_convert_type(atrp_ref[...], jnp.uint32)
    lo = lax.bitcast_convert_type(
        lax.convert_element_type(w & 0xFFFF, jnp.uint16), jnp.bfloat16)
    hi = lax.bitcast_convert_type(
        lax.convert_element_type(w >> 16, jnp.uint16), jnp.bfloat16)
    ar = ar_ref[...]
    dn = (((1,), (1,)), ((), ()))
    o_ref[...] = (
        lax.dot_general(ar[:, :N // 2].astype(jnp.bfloat16), lo, dn,
                        preferred_element_type=jnp.float32)
        + lax.dot_general(ar[:, N // 2:].astype(jnp.bfloat16), hi, dn,
                          preferred_element_type=jnp.float32))


def _pool_matmul(Ar, Atrp):
    g = KP // _MB
    return pl.pallas_call(
        _mm_body,
        grid=(g,),
        in_specs=[
            pl.BlockSpec((_MB, N), lambda i: (i, 0)),
            pl.BlockSpec((KP, N // 2), lambda i: (0, 0)),
        ],
        out_specs=pl.BlockSpec((_MB, KP), lambda i: (i, 0)),
        out_shape=jax.ShapeDtypeStruct((KP, KP), jnp.float32),
    )(Ar, Atrp)


def kernel(X, A, S, kernel):
    feat, idx2d = _head(X, kernel, S)
    idx = jnp.reshape(idx2d, (KP,))
    idx2 = jnp.reshape(idx, (KP // _AC, _AC))
    Xp, Sp2, Ar = _sc_gather_a(feat, A, idx, idx2)
    At = _transpose_packed(A)
    Atr = _sc_gather_at(At, jnp.reshape(idx, (KP // _ACT, _ACT)))
    Ap = _pool_matmul(Ar, Atr)
    Sp = lax.bitcast_convert_type(Sp2[:, 0], jnp.int32)
    return Xp, Ap, Sp
